# speedup vs baseline: 2.9687x; 2.9687x over previous
import functools

import jax
import jax.numpy as jnp
from jax import lax
from jax.experimental import pallas as pl
from jax.experimental.pallas import tpu as pltpu
from jax.experimental.pallas import tpu_sc as plsc

VOCAB = 100000
EMBED_DIM = 128
BATCH = 4096
HIST_LEN = 50

_NC = 2
_NS = 16
_NW = _NC * _NS
_ROWS = BATCH * HIST_LEN
_RPW = _ROWS // _NW
_CHUNK = 128
_NCH = _RPW // _CHUNK

_mesh = plsc.VectorSubcoreMesh(core_axis_name="c", subcore_axis_name="s")


@functools.partial(
    pl.kernel,
    mesh=_mesh,
    out_type=jax.ShapeDtypeStruct((_ROWS, EMBED_DIM), jnp.float32),
    scratch_types=[
        pltpu.VMEM((_NCH, _CHUNK), jnp.int32),
        pltpu.VMEM((2, _CHUNK, EMBED_DIM), jnp.float32),
        pltpu.SemaphoreType.DMA,
    ],
)
def _emb_lookup(idx_hbm, table_hbm, out_hbm, idx_v, rows_v, gsem):
    wid = lax.axis_index("s") * _NC + lax.axis_index("c")
    base = wid * _RPW
    pltpu.sync_copy(idx_hbm.at[wid], idx_v)

    def step(c, carry):
        buf = rows_v.at[0]
        pltpu.async_copy(table_hbm.at[idx_v.at[c]], buf, gsem).wait()
        pltpu.sync_copy(buf, out_hbm.at[pl.ds(base + c * _CHUNK, _CHUNK)])
        return carry

    lax.fori_loop(0, _NCH, step, 0)


def kernel(indices, embedding_matrix):
    idx = indices.reshape(_NW, _NCH, _CHUNK).astype(jnp.int32)
    out = _emb_lookup(idx, embedding_matrix)
    return out.reshape(BATCH, HIST_LEN, EMBED_DIM)


# speedup vs baseline: 3.3287x; 1.1213x over previous
import functools

import jax
import jax.numpy as jnp
from jax import lax
from jax.experimental import pallas as pl
from jax.experimental.pallas import tpu as pltpu
from jax.experimental.pallas import tpu_sc as plsc

VOCAB = 100000
EMBED_DIM = 128
BATCH = 4096
HIST_LEN = 50

_NC = 2
_NS = 16
_NW = _NC * _NS
_ROWS = BATCH * HIST_LEN
_RPW = _ROWS // _NW
_CHUNK = 128
_NCH = _RPW // _CHUNK

_mesh = plsc.VectorSubcoreMesh(core_axis_name="c", subcore_axis_name="s")


_NHALF = _NCH // 2


@functools.partial(
    pl.kernel,
    mesh=_mesh,
    out_type=jax.ShapeDtypeStruct((_ROWS, EMBED_DIM), jnp.float32),
    scratch_types=[
        pltpu.VMEM((_NCH, _CHUNK), jnp.int32),
        pltpu.VMEM((2, _CHUNK, EMBED_DIM), jnp.float32),
        pltpu.SemaphoreType.DMA,
        pltpu.SemaphoreType.DMA,
    ],
)
def _emb_lookup(idx_hbm, table_hbm, out_hbm, idx_v, rows_v, sem0, sem1):
    wid = lax.axis_index("s") * _NC + lax.axis_index("c")
    base = wid * _RPW
    pltpu.sync_copy(idx_hbm.at[wid], idx_v)

    buf0 = rows_v.at[0]
    buf1 = rows_v.at[1]
    pltpu.async_copy(table_hbm.at[idx_v.at[0]], buf0, sem0)

    def step(h, carry):
        c0 = 2 * h
        pltpu.async_copy(table_hbm.at[idx_v.at[c0 + 1]], buf1, sem1)
        pltpu.make_async_copy(table_hbm.at[idx_v.at[0]], buf0, sem0).wait()
        pltpu.sync_copy(buf0, out_hbm.at[pl.ds(base + c0 * _CHUNK, _CHUNK)])
        nxt = lax.min(c0 + 2, _NCH - 2)
        pltpu.async_copy(table_hbm.at[idx_v.at[nxt]], buf0, sem0)
        pltpu.make_async_copy(table_hbm.at[idx_v.at[0]], buf1, sem1).wait()
        pltpu.sync_copy(buf1, out_hbm.at[pl.ds(base + (c0 + 1) * _CHUNK, _CHUNK)])
        return carry

    lax.fori_loop(0, _NHALF, step, 0)
    pltpu.make_async_copy(table_hbm.at[idx_v.at[0]], buf0, sem0).wait()


def kernel(indices, embedding_matrix):
    idx = indices.reshape(_NW, _NCH, _CHUNK).astype(jnp.int32)
    out = _emb_lookup(idx, embedding_matrix)
    return out.reshape(BATCH, HIST_LEN, EMBED_DIM)
